# baseline (speedup 1.0000x reference)
import functools

import jax
import jax.numpy as jnp
from jax import lax
from jax.experimental import pallas as pl
from jax.experimental.pallas import tpu as pltpu
from jax.experimental.pallas import tpu_sc as plsc

_B, _L, _D = 1024, 200, 128
_NLANE = 16
_ROW = _L * _D
_RPW = _B // 32
_TPW = _RPW * _L
_NSLOT = 4
_UNITS = [(r, off, n) for r in range(_RPW) for off, n in ((0, 104), (104, 96))]


def _ct_body(tab_ref, pe_ref, out_ref):
    out_ref[...] = tab_ref[...][:, None, :] + pe_ref[...][None, :, :]


def _build_ct(seg_table, pe):
    ct = pl.pallas_call(
        _ct_body,
        out_shape=jax.ShapeDtypeStruct((3, _L, _D), jnp.float32),
    )(seg_table, pe)
    return ct.reshape(3 * _L, _D)


def _sc_body(
    seq_hbm, lab_hbm, ct_hbm, out_hbm,
    idx_all, seq0, seq1, seq2, seq3, seg0, seg1, seg2, seg3, ct_sh,
    sem_lab, sg0, sg1, sg2, sg3, ss0, ss1, ss2, ss3, so0, so1, so2, so3,
):
    nc = 2
    sid = lax.axis_index("s")
    wid = sid * nc + lax.axis_index("c")
    tok_base = wid * _TPW

    seqs = (seq0, seq1, seq2, seq3)
    segs = (seg0, seg1, seg2, seg3)
    sem_g = (sg0, sg1, sg2, sg3)
    sem_seq = (ss0, ss1, ss2, ss3)
    sem_out = (so0, so1, so2, so3)

    lab_cp = pltpu.make_async_copy(
        lab_hbm.at[pl.ds(tok_base, _TPW)], idx_all, sem_lab
    )
    lab_cp.start()

    @pl.when(sid == 0)
    def _():
        pltpu.sync_copy(ct_hbm, ct_sh)

    plsc.subcore_barrier()
    lab_cp.wait()

    def idx_body(g, c):
        sl = pl.ds(g * _NLANE, _NLANE)
        tvec = g * _NLANE + lax.iota(jnp.int32, _NLANE)
        idx_all[sl] = idx_all[sl] * _L + lax.rem(tvec, _L)
        return c

    lax.fori_loop(0, _TPW // _NLANE, idx_body, 0, unroll=4)

    def g_cp(u, s):
        r, off, n = _UNITS[u]
        t0 = r * _L + off
        return pltpu.make_async_copy(
            ct_sh.at[idx_all.at[pl.ds(t0, n)]], segs[s].at[pl.ds(0, n)], sem_g[s]
        )

    def seq_cp(u, s):
        r, off, n = _UNITS[u]
        w0 = (r * _L + off) * _D
        return pltpu.make_async_copy(
            seq_hbm.at[pl.ds(tok_base * _D + w0, n * _D)],
            seqs[s].at[pl.ds(0, n * _D)],
            sem_seq[s],
        )

    def out_cp(u, s):
        r, off, n = _UNITS[u]
        w0 = (r * _L + off) * _D
        return pltpu.make_async_copy(
            seqs[s].at[pl.ds(0, n * _D)],
            out_hbm.at[pl.ds(tok_base * _D + w0, n * _D)],
            sem_out[s],
        )

    def start_unit(u):
        s = u % _NSLOT
        g_cp(u, s).start()
        seq_cp(u, s).start()

    def add(u, s):
        n = _UNITS[u][2]

        def body(t, c):
            loff = t * _D
            for d in range(_D // _NLANE):
                plsc.addupdate(
                    seqs[s].at[pl.ds(loff + d * _NLANE, _NLANE)],
                    segs[s][t, pl.ds(d * _NLANE, _NLANE)],
                )
            return c

        lax.fori_loop(0, n, body, 0, unroll=2)

    nu = len(_UNITS)
    for u in range(_NSLOT - 1):
        start_unit(u)
    for u in range(nu):
        s = u % _NSLOT
        g_cp(u, s).wait()
        seq_cp(u, s).wait()
        if u >= 1:
            out_cp(u - 1, (u - 1) % _NSLOT).wait()
        if u + _NSLOT - 1 < nu:
            start_unit(u + _NSLOT - 1)
        add(u, s)
        out_cp(u, s).start()
    out_cp(nu - 1, (nu - 1) % _NSLOT).wait()


def kernel(sequence, segment_label, seg_table, pos_pe):
    pe = pos_pe.reshape(_L, _D)
    ct = _build_ct(seg_table, pe)

    mesh = plsc.VectorSubcoreMesh(core_axis_name="c", subcore_axis_name="s")
    k = functools.partial(
        pl.kernel,
        mesh=mesh,
        out_type=jax.ShapeDtypeStruct((_B * _L * _D,), jnp.float32),
        scratch_types=[
            pltpu.VMEM((_TPW,), jnp.int32),
            pltpu.VMEM((104 * _D,), jnp.float32),
            pltpu.VMEM((104 * _D,), jnp.float32),
            pltpu.VMEM((104 * _D,), jnp.float32),
            pltpu.VMEM((104 * _D,), jnp.float32),
            pltpu.VMEM((104, _D), jnp.float32),
            pltpu.VMEM((104, _D), jnp.float32),
            pltpu.VMEM((104, _D), jnp.float32),
            pltpu.VMEM((104, _D), jnp.float32),
            pltpu.VMEM_SHARED((3 * _L, _D), jnp.float32),
            pltpu.SemaphoreType.DMA,
            pltpu.SemaphoreType.DMA,
            pltpu.SemaphoreType.DMA,
            pltpu.SemaphoreType.DMA,
            pltpu.SemaphoreType.DMA,
            pltpu.SemaphoreType.DMA,
            pltpu.SemaphoreType.DMA,
            pltpu.SemaphoreType.DMA,
            pltpu.SemaphoreType.DMA,
            pltpu.SemaphoreType.DMA,
            pltpu.SemaphoreType.DMA,
            pltpu.SemaphoreType.DMA,
            pltpu.SemaphoreType.DMA,
        ],
    )(_sc_body)
    out = k(sequence.reshape(-1), segment_label.reshape(-1), ct)
    return out.reshape(_B, _L, _D)


# speedup vs baseline: 1.0471x; 1.0471x over previous
import functools

import jax
import jax.numpy as jnp
from jax import lax
from jax.experimental import pallas as pl
from jax.experimental.pallas import tpu as pltpu
from jax.experimental.pallas import tpu_sc as plsc

_B, _L, _D = 1024, 200, 128
_NLANE = 16
_NVREG_L = 13
_ROW = _L * _D
_RPW = _B // 32


def _ct_body(tab_ref, pe_ref, out_ref):
    out_ref[...] = tab_ref[...][:, None, :] + pe_ref[...][None, :, :]


def _build_ct(seg_table, pe):
    ct = pl.pallas_call(
        _ct_body,
        out_shape=jax.ShapeDtypeStruct((3, _L, _D), jnp.float32),
    )(seg_table, pe)
    return ct.reshape(3 * _L, _D)


def _sc_body(
    seq_hbm, lab_hbm, ct_hbm, out_hbm,
    seq0, seq1, seg0, seg1, idx0, idx1, ct_sh,
    sl0, sl1, sg0, sg1, ss0, ss1, so0, so1,
):
    nc = 2
    sid = lax.axis_index("s")
    wid = sid * nc + lax.axis_index("c")
    base = wid * _RPW

    @pl.when(sid == 0)
    def _():
        pltpu.sync_copy(ct_hbm, ct_sh)

    plsc.subcore_barrier()

    seqs = (seq0, seq1)
    segs = (seg0, seg1)
    idxs = (idx0, idx1)
    sem_lab = (sl0, sl1)
    sem_g = (sg0, sg1)
    sem_seq = (ss0, ss1)
    sem_out = (so0, so1)

    def lab_cp(b, s):
        return pltpu.make_async_copy(
            lab_hbm.at[pl.ds(b * _L, _L)], idxs[s].at[pl.ds(0, _L)], sem_lab[s]
        )

    def g_cp0(s):
        return pltpu.make_async_copy(
            ct_sh.at[idxs[s].at[pl.ds(0, 104)]], segs[s].at[pl.ds(0, 104)], sem_g[s]
        )

    def g_cp1(s):
        return pltpu.make_async_copy(
            ct_sh.at[idxs[s].at[pl.ds(104, 96)]], segs[s].at[pl.ds(104, 96)], sem_g[s]
        )

    def seq_cp(b, s):
        return pltpu.make_async_copy(
            seq_hbm.at[pl.ds(b * _ROW, _ROW)], seqs[s], sem_seq[s]
        )

    def out_cp(b, s):
        return pltpu.make_async_copy(
            seqs[s], out_hbm.at[pl.ds(b * _ROW, _ROW)], sem_out[s]
        )

    def stage_a(b, s):
        lab_cp(b, s).start()

    def stage_b(b, s):
        lab_cp(b, s).wait()
        for j in range(_NVREG_L):
            sl = pl.ds(j * _NLANE, _NLANE)
            pos = j * _NLANE + lax.iota(jnp.int32, _NLANE)
            idxs[s][sl] = idxs[s][sl] * _L + pos
        g_cp0(s).start()
        g_cp1(s).start()
        seq_cp(b, s).start()

    def stage_c(b, s):
        g_cp0(s).wait()
        g_cp1(s).wait()
        seq_cp(b, s).wait()

        def add_body(l, c):
            off = l * _D
            for d in range(_D // _NLANE):
                plsc.addupdate(
                    seqs[s].at[pl.ds(off + d * _NLANE, _NLANE)],
                    segs[s][l, pl.ds(d * _NLANE, _NLANE)],
                )
            return c

        lax.fori_loop(0, _L, add_body, 0, unroll=2)
        out_cp(b, s).start()

    stage_a(base, 0)
    stage_b(base, 0)
    stage_a(base + 1, 1)
    for r in range(_RPW):
        s = r & 1
        if r >= 1:
            out_cp(base + r - 1, 1 - s).wait()
        if r + 1 < _RPW:
            stage_b(base + r + 1, 1 - s)
        stage_c(base + r, s)
        if r + 2 < _RPW:
            stage_a(base + r + 2, s)
    out_cp(base + _RPW - 1, 1).wait()


def kernel(sequence, segment_label, seg_table, pos_pe):
    pe = pos_pe.reshape(_L, _D)
    ct = _build_ct(seg_table, pe)

    mesh = plsc.VectorSubcoreMesh(core_axis_name="c", subcore_axis_name="s")
    k = functools.partial(
        pl.kernel,
        mesh=mesh,
        out_type=jax.ShapeDtypeStruct((_B * _L * _D,), jnp.float32),
        scratch_types=[
            pltpu.VMEM((_ROW,), jnp.float32),
            pltpu.VMEM((_ROW,), jnp.float32),
            pltpu.VMEM((_L, _D), jnp.float32),
            pltpu.VMEM((_L, _D), jnp.float32),
            pltpu.VMEM((208,), jnp.int32),
            pltpu.VMEM((208,), jnp.int32),
            pltpu.VMEM_SHARED((3 * _L, _D), jnp.float32),
            pltpu.SemaphoreType.DMA,
            pltpu.SemaphoreType.DMA,
            pltpu.SemaphoreType.DMA,
            pltpu.SemaphoreType.DMA,
            pltpu.SemaphoreType.DMA,
            pltpu.SemaphoreType.DMA,
            pltpu.SemaphoreType.DMA,
            pltpu.SemaphoreType.DMA,
        ],
    )(_sc_body)
    out = k(sequence.reshape(-1), segment_label.reshape(-1), ct)
    return out.reshape(_B, _L, _D)
